# Initial kernel scaffold; baseline (speedup 1.0000x reference)
import jax
import jax.numpy as jnp
from jax.experimental import pallas as pl


def kernel(input_):
    raise NotImplementedError("write your pallas kernel here")


# speedup vs baseline: 6.9570x; 6.9570x over previous
import jax
import jax.numpy as jnp
from jax import lax
from jax.experimental import pallas as pl
from jax.experimental.pallas import tpu as pltpu

K = 16
ROWS_PER_BLOCK = 8


def _tc_body(x_ref, o_ref):
    x = x_ref[...]
    r, l = x.shape
    x3 = x.reshape(r, l // K, K)
    a = jnp.abs(x3)
    m = jnp.max(a, axis=-1, keepdims=True)
    idx = lax.broadcasted_iota(jnp.int32, x3.shape, 2)
    cand = jnp.where(a >= m, idx, K)
    first = jnp.min(cand, axis=-1, keepdims=True)
    out = jnp.where(idx == first, x3, 0.0)
    o_ref[...] = out.reshape(r, l)


def kernel(input_):
    b, c, l = input_.shape
    rows = b * c
    x2 = input_.reshape(rows, l)
    r = ROWS_PER_BLOCK
    out = pl.pallas_call(
        _tc_body,
        grid=(rows // r,),
        in_specs=[pl.BlockSpec((r, l), lambda i: (i, 0))],
        out_specs=pl.BlockSpec((r, l), lambda i: (i, 0)),
        out_shape=jax.ShapeDtypeStruct((rows, l), input_.dtype),
    )(x2)
    return out.reshape(b, c, l)


# speedup vs baseline: 17.4883x; 2.5138x over previous
import jax
import jax.numpy as jnp
from jax import lax
from jax.experimental import pallas as pl

K = 16
ROWS_PER_BLOCK = 32


def _suffix_reduce(v, op):
    for s in (1, 2, 4, 8):
        v = op(v, jnp.roll(v, -s, axis=-1))
    return v


def _leader_broadcast(v, lane):
    for s in (1, 2, 4, 8):
        v = jnp.where((lane & s) != 0, jnp.roll(v, s, axis=-1), v)
    return v


def _tc_body(x_ref, o_ref):
    x = x_ref[...]
    r, l = x.shape
    x2 = x.reshape(r * l // 128, 128)
    lane = lax.broadcasted_iota(jnp.int32, x2.shape, 1)
    j = (lane % K).astype(jnp.float32)

    a = jnp.abs(x2)
    m = _suffix_reduce(a, jnp.maximum)
    g = _leader_broadcast(m, lane)
    cand = jnp.where(a >= g, j, float(K))
    cm = _suffix_reduce(cand, jnp.minimum)
    first = _leader_broadcast(cm, lane)
    out = jnp.where(j == first, x2, 0.0)
    o_ref[...] = out.reshape(r, l)


def kernel(input_):
    b, c, l = input_.shape
    rows = b * c
    x2 = input_.reshape(rows, l)
    r = min(ROWS_PER_BLOCK, rows)
    out = pl.pallas_call(
        _tc_body,
        grid=(rows // r,),
        in_specs=[pl.BlockSpec((r, l), lambda i: (i, 0))],
        out_specs=pl.BlockSpec((r, l), lambda i: (i, 0)),
        out_shape=jax.ShapeDtypeStruct((rows, l), input_.dtype),
    )(x2)
    return out.reshape(b, c, l)
